# baseline (device time: 62275 ns/iter reference)
import functools

import jax
import jax.numpy as jnp
from jax import lax
from jax.experimental import pallas as pl
from jax.experimental.pallas import tpu as pltpu

T = 2048
D = 1024
R = 64
MAX_CHUNKS = T // R


def kernel(x, dest):
    my_y = lax.axis_index("y")
    order = jnp.argsort(dest, stable=True)
    s = x.astype(jnp.bfloat16)[order].reshape(T, 8, 128)

    c0 = jnp.sum((dest == 0).astype(jnp.int32))
    K = jnp.where(my_y == 0, T - c0, c0)
    base = jnp.where(my_y == 0, c0, 0)
    doff = jnp.where(my_y == 0, 0, T - K)
    klo = jnp.where(my_y == 0, 0, c0)
    kn = T - K
    nc = (K + R - 1) // R
    nk = (kn + R - 1) // R
    scal = jnp.stack([nc, base, doff, K, klo, kn, nk]).astype(jnp.int32)

    def body(scal_ref, s_ref, out_ref, send_sems, recv_sems, loc_sems, ready_sem):
        mx = lax.axis_index("x")
        my = lax.axis_index("y")
        mz = lax.axis_index("z")
        nbr = (mx, 1 - my, mz)
        nc_ = scal_ref[0]
        base_ = scal_ref[1]
        doff_ = scal_ref[2]
        k_ = scal_ref[3]
        klo_ = scal_ref[4]
        kn_ = scal_ref[5]
        nk_ = scal_ref[6]

        for j in range(MAX_CHUNKS):
            @pl.when(j < nk_)
            def _():
                o = jnp.maximum(jnp.minimum(j * R, kn_ - R), 0)
                pltpu.make_async_copy(
                    s_ref.at[pl.ds(klo_ + o, R), :, :],
                    out_ref.at[pl.ds(klo_ + o, R), :, :],
                    loc_sems.at[j],
                ).start()

        for j in range(MAX_CHUNKS):
            @pl.when(j < nk_)
            def _():
                o = jnp.maximum(jnp.minimum(j * R, kn_ - R), 0)
                pltpu.make_async_copy(
                    s_ref.at[pl.ds(klo_ + o, R), :, :],
                    out_ref.at[pl.ds(klo_ + o, R), :, :],
                    loc_sems.at[j],
                ).wait()

        pl.semaphore_signal(
            ready_sem, inc=1, device_id=nbr,
            device_id_type=pl.DeviceIdType.MESH,
        )
        pl.semaphore_wait(ready_sem, 1)

        for j in range(MAX_CHUNKS):
            @pl.when(j < nc_)
            def _():
                o = jnp.maximum(jnp.minimum(j * R, k_ - R), 0)
                rdma = pltpu.make_async_remote_copy(
                    src_ref=s_ref.at[pl.ds(base_ + o, R), :, :],
                    dst_ref=out_ref.at[pl.ds(doff_ + o, R), :, :],
                    send_sem=send_sems.at[j],
                    recv_sem=recv_sems.at[j],
                    device_id=nbr,
                    device_id_type=pl.DeviceIdType.MESH,
                )
                rdma.start()

        for j in range(MAX_CHUNKS):
            @pl.when(j < nc_)
            def _():
                o = jnp.maximum(jnp.minimum(j * R, k_ - R), 0)
                rdma = pltpu.make_async_remote_copy(
                    src_ref=s_ref.at[pl.ds(base_ + o, R), :, :],
                    dst_ref=out_ref.at[pl.ds(doff_ + o, R), :, :],
                    send_sem=send_sems.at[j],
                    recv_sem=recv_sems.at[j],
                    device_id=nbr,
                    device_id_type=pl.DeviceIdType.MESH,
                )
                rdma.wait()

    grid_body = functools.partial(
        pl.run_scoped,
        ready_sem=pltpu.SemaphoreType.REGULAR,
    )

    def wrapped_body(scal_ref, s_ref, out_ref, send_sems, recv_sems, loc_sems):
        grid_body(
            lambda ready_sem: body(
                scal_ref, s_ref, out_ref, send_sems, recv_sems, loc_sems,
                ready_sem,
            )
        )

    return pl.pallas_call(
        wrapped_body,
        out_shape=jax.ShapeDtypeStruct((T, 8, 128), jnp.bfloat16),
        in_specs=[
            pl.BlockSpec(memory_space=pltpu.SMEM),
            pl.BlockSpec(memory_space=pltpu.VMEM),
        ],
        out_specs=pl.BlockSpec(memory_space=pltpu.VMEM),
        scratch_shapes=[
            pltpu.SemaphoreType.DMA((MAX_CHUNKS,)),
            pltpu.SemaphoreType.DMA((MAX_CHUNKS,)),
            pltpu.SemaphoreType.DMA((MAX_CHUNKS,)),
        ],
    )(scal, s).reshape(T, D)


# device time: 61584 ns/iter; 1.0112x vs baseline; 1.0112x over previous
import jax
import jax.numpy as jnp
from jax import lax
from jax.experimental import pallas as pl
from jax.experimental.pallas import tpu as pltpu

T = 2048
D = 1024
R = 64
MAX_CHUNKS = T // R


def kernel(x, dest):
    my_y = lax.axis_index("y")
    d0 = (dest == 0).astype(jnp.int32)
    cz = jnp.cumsum(d0)
    c0 = cz[T - 1]
    i32 = jnp.arange(T, dtype=jnp.int32)
    pos = jnp.where(d0 == 1, cz - 1, c0 + i32 - cz)
    order = jnp.zeros(T, jnp.int32).at[pos].set(i32, unique_indices=True)
    s = x.astype(jnp.bfloat16).reshape(T, 8, 128)[order]
    K = jnp.where(my_y == 0, T - c0, c0)
    base = jnp.where(my_y == 0, c0, 0)
    doff = jnp.where(my_y == 0, 0, T - K)
    klo = jnp.where(my_y == 0, 0, c0)
    kn = T - K
    nc = (K + R - 1) // R
    scal = jnp.stack([nc, base, doff, K, klo, kn]).astype(jnp.int32)

    def body(scal_ref, s_ref, out_ref, send_sems, recv_sems):
        mx = lax.axis_index("x")
        my = lax.axis_index("y")
        mz = lax.axis_index("z")
        nbr = (mx, 1 - my, mz)
        nc_ = scal_ref[0]
        base_ = scal_ref[1]
        doff_ = scal_ref[2]
        k_ = scal_ref[3]
        klo_ = scal_ref[4]
        kn_ = scal_ref[5]

        for j in range(MAX_CHUNKS):
            @pl.when(j < nc_)
            def _():
                o = jnp.maximum(jnp.minimum(j * R, k_ - R), 0)
                rdma = pltpu.make_async_remote_copy(
                    src_ref=s_ref.at[pl.ds(base_ + o, R), :, :],
                    dst_ref=out_ref.at[pl.ds(doff_ + o, R), :, :],
                    send_sem=send_sems.at[j],
                    recv_sem=recv_sems.at[j],
                    device_id=nbr,
                    device_id_type=pl.DeviceIdType.MESH,
                )
                rdma.start()

        for j in range(MAX_CHUNKS):
            @pl.when(j < nc_)
            def _():
                o = jnp.maximum(jnp.minimum(j * R, k_ - R), 0)
                rdma = pltpu.make_async_remote_copy(
                    src_ref=s_ref.at[pl.ds(base_ + o, R), :, :],
                    dst_ref=out_ref.at[pl.ds(doff_ + o, R), :, :],
                    send_sem=send_sems.at[j],
                    recv_sem=recv_sems.at[j],
                    device_id=nbr,
                    device_id_type=pl.DeviceIdType.MESH,
                )
                rdma.wait()

        rows = lax.broadcasted_iota(jnp.int32, (T, 8, 128), 0)
        kept = (rows >= klo_) & (rows < klo_ + kn_)
        out_ref[:, :, :] = jnp.where(kept, s_ref[:, :, :], out_ref[:, :, :])

    return pl.pallas_call(
        body,
        out_shape=jax.ShapeDtypeStruct((T, 8, 128), jnp.bfloat16),
        in_specs=[
            pl.BlockSpec(memory_space=pltpu.SMEM),
            pl.BlockSpec(memory_space=pltpu.VMEM),
        ],
        out_specs=pl.BlockSpec(memory_space=pltpu.VMEM),
        scratch_shapes=[
            pltpu.SemaphoreType.DMA((MAX_CHUNKS,)),
            pltpu.SemaphoreType.DMA((MAX_CHUNKS,)),
        ],
    )(scal, s).reshape(T, D)


# device time: 59541 ns/iter; 1.0459x vs baseline; 1.0343x over previous
import jax
import jax.numpy as jnp
from jax import lax
from jax.experimental import pallas as pl
from jax.experimental.pallas import tpu as pltpu

T = 2048
D = 1024
R = 256
MAX_CHUNKS = T // R


def kernel(x, dest):
    my_y = lax.axis_index("y")
    order = jnp.argsort(dest, stable=True)
    s = x.astype(jnp.bfloat16)[order].reshape(T, 8, 128)

    c0 = jnp.sum((dest == 0).astype(jnp.int32))
    K = jnp.where(my_y == 0, T - c0, c0)
    base = jnp.where(my_y == 0, c0, 0)
    doff = jnp.where(my_y == 0, 0, T - K)
    klo = jnp.where(my_y == 0, 0, c0)
    kn = T - K
    nc = (K + R - 1) // R
    scal = jnp.stack([nc, base, doff, K, klo, kn]).astype(jnp.int32)

    def body(scal_ref, s_ref, out_ref, send_sems, recv_sems):
        mx = lax.axis_index("x")
        my = lax.axis_index("y")
        mz = lax.axis_index("z")
        nbr = (mx, 1 - my, mz)
        nc_ = scal_ref[0]
        base_ = scal_ref[1]
        doff_ = scal_ref[2]
        k_ = scal_ref[3]
        klo_ = scal_ref[4]
        kn_ = scal_ref[5]

        for j in range(MAX_CHUNKS):
            @pl.when(j < nc_)
            def _():
                o = jnp.maximum(jnp.minimum(j * R, k_ - R), 0)
                rdma = pltpu.make_async_remote_copy(
                    src_ref=s_ref.at[pl.ds(base_ + o, R), :, :],
                    dst_ref=out_ref.at[pl.ds(doff_ + o, R), :, :],
                    send_sem=send_sems.at[j],
                    recv_sem=recv_sems.at[j],
                    device_id=nbr,
                    device_id_type=pl.DeviceIdType.MESH,
                )
                rdma.start()

        for j in range(MAX_CHUNKS):
            @pl.when(j < nc_)
            def _():
                o = jnp.maximum(jnp.minimum(j * R, k_ - R), 0)
                rdma = pltpu.make_async_remote_copy(
                    src_ref=s_ref.at[pl.ds(base_ + o, R), :, :],
                    dst_ref=out_ref.at[pl.ds(doff_ + o, R), :, :],
                    send_sem=send_sems.at[j],
                    recv_sem=recv_sems.at[j],
                    device_id=nbr,
                    device_id_type=pl.DeviceIdType.MESH,
                )
                rdma.wait()

        rows = lax.broadcasted_iota(jnp.int32, (T, 1, 1), 0)
        kept = (rows >= klo_) & (rows < klo_ + kn_)
        out_ref[:, :, :] = jnp.where(kept, s_ref[:, :, :], out_ref[:, :, :])

    return pl.pallas_call(
        body,
        out_shape=jax.ShapeDtypeStruct((T, 8, 128), jnp.bfloat16),
        in_specs=[
            pl.BlockSpec(memory_space=pltpu.SMEM),
            pl.BlockSpec(memory_space=pltpu.VMEM),
        ],
        out_specs=pl.BlockSpec(memory_space=pltpu.VMEM),
        scratch_shapes=[
            pltpu.SemaphoreType.DMA((MAX_CHUNKS,)),
            pltpu.SemaphoreType.DMA((MAX_CHUNKS,)),
        ],
    )(scal, s).reshape(T, D)
